# Initial kernel scaffold; baseline (speedup 1.0000x reference)
import jax
import jax.numpy as jnp
from jax.experimental import pallas as pl


def kernel(h, edge_attr, emb_w, emb_b, edge_w1, edge_b1, edge_w2, edge_b2, node_w1, node_b1, node_w2, node_b2, out_w, out_b, edge_index):
    raise NotImplementedError("write your pallas kernel here")


# speedup vs baseline: 1.4158x; 1.4158x over previous
import functools

import jax
import jax.numpy as jnp
from jax import lax
from jax.experimental import pallas as pl
from jax.experimental.pallas import tpu as pltpu
from jax.experimental.pallas import tpu_sc as plsc

N = 10000
E = 160000
IN_NF = 128
EDGE_NF = 16
HID = 256
OUT_NF = 128
L = 4
EAP = 128

NC = 2
NS = 16
LANES = 16
NW = NC * NS

BN = 1000
BE = 2000
CH_G = 40
CH_S = 80
HALF = N // NC
ROWS_T = 320
ACC_R = NS * ROWS_T
TRASH = ACC_R - 1


def _silu(x):
    return x * lax.logistic(x)


def _embed_pre_body(h_ref, ew_ref, eb_ref, w1s_ref, w1t_ref, b1_ref,
                    x_ref, xa_ref, xb_ref):
    x = jnp.dot(h_ref[...], ew_ref[...], preferred_element_type=jnp.float32)
    x = x + eb_ref[...]
    x_ref[...] = x
    xa_ref[...] = jnp.dot(x, w1s_ref[...], preferred_element_type=jnp.float32)
    xb_ref[...] = (jnp.dot(x, w1t_ref[...], preferred_element_type=jnp.float32)
                   + b1_ref[...])


def _embed_pre(h, emb_w, emb_b, w1s, w1t, b1):
    grid = (N // BN,)
    full = lambda r, c: pl.BlockSpec((r, c), lambda i: (0, 0))
    return pl.pallas_call(
        _embed_pre_body,
        grid=grid,
        in_specs=[
            pl.BlockSpec((BN, IN_NF), lambda i: (i, 0)),
            full(IN_NF, HID),
            full(1, HID),
            full(HID, HID),
            full(HID, HID),
            full(1, HID),
        ],
        out_specs=[
            pl.BlockSpec((BN, HID), lambda i: (i, 0)),
            pl.BlockSpec((BN, HID), lambda i: (i, 0)),
            pl.BlockSpec((BN, HID), lambda i: (i, 0)),
        ],
        out_shape=[
            jax.ShapeDtypeStruct((N, HID), jnp.float32),
            jax.ShapeDtypeStruct((N, HID), jnp.float32),
            jax.ShapeDtypeStruct((N, HID), jnp.float32),
        ],
    )(h, emb_w, emb_b, w1s, w1t, b1)


def _edge_mlp_body(g_ref, ea_ref, w1e_ref, w2_ref, b2_ref, v_ref):
    u = g_ref[...] + jnp.dot(ea_ref[...], w1e_ref[...],
                             preferred_element_type=jnp.float32)
    u = _silu(u)
    v = jnp.dot(u, w2_ref[...], preferred_element_type=jnp.float32) + b2_ref[...]
    v_ref[...] = _silu(v)


def _edge_mlp(g, edge_attr, w1e, w2, b2):
    grid = (E // BE,)
    full = lambda r, c: pl.BlockSpec((r, c), lambda i: (0, 0))
    return pl.pallas_call(
        _edge_mlp_body,
        grid=grid,
        in_specs=[
            pl.BlockSpec((BE, HID), lambda i: (i, 0)),
            pl.BlockSpec((BE, EAP), lambda i: (i, 0)),
            full(EAP, HID),
            full(HID, HID),
            full(1, HID),
        ],
        out_specs=pl.BlockSpec((BE, HID), lambda i: (i, 0)),
        out_shape=jax.ShapeDtypeStruct((E, HID), jnp.float32),
    )(g, edge_attr, w1e, w2, b2)


def _node_pre_body(x_ref, agg_ref, w1x_ref, w1a_ref, b1_ref, w2_ref, b2_ref,
                   w1s_ref, w1t_ref, b1e_ref, xn_ref, xa_ref, xb_ref):
    u = (jnp.dot(x_ref[...], w1x_ref[...], preferred_element_type=jnp.float32)
         + jnp.dot(agg_ref[...], w1a_ref[...], preferred_element_type=jnp.float32)
         + b1_ref[...])
    u = _silu(u)
    o = jnp.dot(u, w2_ref[...], preferred_element_type=jnp.float32) + b2_ref[...]
    xn = x_ref[...] + o
    xn_ref[...] = xn
    xa_ref[...] = jnp.dot(xn, w1s_ref[...], preferred_element_type=jnp.float32)
    xb_ref[...] = (jnp.dot(xn, w1t_ref[...], preferred_element_type=jnp.float32)
                   + b1e_ref[...])


def _node_pre(x, agg, w1x, w1a, b1, w2, b2, w1s, w1t, b1e):
    grid = (N // BN,)
    full = lambda r, c: pl.BlockSpec((r, c), lambda i: (0, 0))
    return pl.pallas_call(
        _node_pre_body,
        grid=grid,
        in_specs=[
            pl.BlockSpec((BN, HID), lambda i: (i, 0)),
            pl.BlockSpec((BN, HID), lambda i: (i, 0)),
            full(HID, HID), full(HID, HID), full(1, HID),
            full(HID, HID), full(1, HID),
            full(HID, HID), full(HID, HID), full(1, HID),
        ],
        out_specs=[
            pl.BlockSpec((BN, HID), lambda i: (i, 0)),
            pl.BlockSpec((BN, HID), lambda i: (i, 0)),
            pl.BlockSpec((BN, HID), lambda i: (i, 0)),
        ],
        out_shape=[
            jax.ShapeDtypeStruct((N, HID), jnp.float32),
            jax.ShapeDtypeStruct((N, HID), jnp.float32),
            jax.ShapeDtypeStruct((N, HID), jnp.float32),
        ],
    )(x, agg, w1x, w1a, b1, w2, b2, w1s, w1t, b1e)


def _node_final_body(x_ref, agg_ref, w1x_ref, w1a_ref, b1_ref, w2_ref, b2_ref,
                     ow_ref, ob_ref, out_ref):
    u = (jnp.dot(x_ref[...], w1x_ref[...], preferred_element_type=jnp.float32)
         + jnp.dot(agg_ref[...], w1a_ref[...], preferred_element_type=jnp.float32)
         + b1_ref[...])
    u = _silu(u)
    o = jnp.dot(u, w2_ref[...], preferred_element_type=jnp.float32) + b2_ref[...]
    xn = x_ref[...] + o
    out_ref[...] = (jnp.dot(xn, ow_ref[...], preferred_element_type=jnp.float32)
                    + ob_ref[...])


def _node_final(x, agg, w1x, w1a, b1, w2, b2, out_w, out_b):
    grid = (N // BN,)
    full = lambda r, c: pl.BlockSpec((r, c), lambda i: (0, 0))
    return pl.pallas_call(
        _node_final_body,
        grid=grid,
        in_specs=[
            pl.BlockSpec((BN, HID), lambda i: (i, 0)),
            pl.BlockSpec((BN, HID), lambda i: (i, 0)),
            full(HID, HID), full(HID, HID), full(1, HID),
            full(HID, HID), full(1, HID),
            full(HID, OUT_NF), full(1, OUT_NF),
        ],
        out_specs=pl.BlockSpec((BN, OUT_NF), lambda i: (i, 0)),
        out_shape=jax.ShapeDtypeStruct((N, OUT_NF), jnp.float32),
    )(x, agg, w1x, w1a, b1, w2, b2, out_w, out_b)


def _sc_mesh():
    return plsc.VectorSubcoreMesh(core_axis_name="c", subcore_axis_name="s",
                                  num_cores=NC, num_subcores=NS)


_SC_PARAMS = pltpu.CompilerParams(needs_layout_passes=False)


def _gather_add_body(xa_hbm, xb_hbm, row_hbm, col_hbm, g_hbm,
                     ridx, cidx, abuf, bbuf, sem):
    wid = lax.axis_index("s") * NC + lax.axis_index("c")
    epw = E // NW
    base = wid * epw

    def body(ei, carry):
        off = base + ei * CH_G
        pltpu.sync_copy(row_hbm.at[pl.ds(off, CH_G)], ridx)
        pltpu.sync_copy(col_hbm.at[pl.ds(off, CH_G)], cidx)
        ca = pltpu.async_copy(xa_hbm.at[ridx], abuf, sem)
        cb = pltpu.async_copy(xb_hbm.at[cidx], bbuf, sem)
        ca.wait()
        cb.wait()

        def add_row(r, c2):
            for j in range(HID // LANES):
                sl = pl.ds(j * LANES, LANES)
                abuf[r, sl] = abuf[r, sl] + bbuf[r, sl]
            return c2

        lax.fori_loop(0, CH_G, add_row, 0)
        pltpu.sync_copy(abuf, g_hbm.at[pl.ds(off, CH_G)])
        return carry

    lax.fori_loop(0, epw // CH_G, body, 0)


def _gather_add(xa, xb, row, col):
    return pl.kernel(
        _gather_add_body,
        out_type=jax.ShapeDtypeStruct((E, HID), jnp.float32),
        mesh=_sc_mesh(),
        compiler_params=_SC_PARAMS,
        scratch_types=[
            pltpu.VMEM((CH_G,), jnp.int32),
            pltpu.VMEM((CH_G,), jnp.int32),
            pltpu.VMEM((CH_G, HID), jnp.float32),
            pltpu.VMEM((CH_G, HID), jnp.float32),
            pltpu.SemaphoreType.DMA,
        ],
    )(xa, xb, row, col)


def _permute_body(ea_hbm, perm_hbm, out_hbm, pidx, ebuf, sem):
    wid = lax.axis_index("c") * NS + lax.axis_index("s")
    epw = E // NW
    base = wid * epw

    def body(ei, carry):
        off = base + ei * CH_G
        pltpu.sync_copy(perm_hbm.at[pl.ds(off, CH_G)], pidx)
        pltpu.async_copy(ea_hbm.at[pidx], ebuf, sem).wait()
        pltpu.sync_copy(ebuf, out_hbm.at[pl.ds(off, CH_G)])
        return carry

    lax.fori_loop(0, epw // CH_G, body, 0)


def _permute_rows(edge_attr, perm):
    return pl.kernel(
        _permute_body,
        out_type=jax.ShapeDtypeStruct((E, EAP), jnp.float32),
        mesh=_sc_mesh(),
        compiler_params=_SC_PARAMS,
        scratch_types=[
            pltpu.VMEM((CH_G,), jnp.int32),
            pltpu.VMEM((CH_G, EAP), jnp.float32),
            pltpu.SemaphoreType.DMA,
        ],
    )(edge_attr, perm)


TN = 312
TN_LAST = N - (NW - 1) * TN
CH_V = 80


def _lane_extract(vec, lane):
    io = lax.iota(jnp.int32, LANES)
    return jnp.sum(jnp.where(io == lane, vec, 0))


def _scatter_seg_body(v_hbm, row_hbm, eb_hbm, agg_hbm, ridx, vbuf, bvec, acc):
    wid = lax.axis_index("c") * NS + lax.axis_index("s")
    node_lo = wid * TN

    pltpu.sync_copy(eb_hbm, bvec)

    def bound(i):
        sub = (i // LANES) * LANES
        return _lane_extract(bvec[pl.ds(sub, LANES)], i - sub)

    lo = bound(wid)
    hi = bound(wid + 1)

    zz = jnp.zeros((LANES,), jnp.float32)

    def zrow(r, c2):
        for j in range(HID // LANES):
            acc[r, pl.ds(j * LANES, LANES)] = zz
        return c2

    lax.fori_loop(0, TN_LAST, zrow, 0)

    base0 = (lo // 8) * 8
    nch = (hi - base0 + CH_V - 1) // CH_V

    def flush(cur, regs):
        @pl.when(cur >= 0)
        def _():
            for j in range(HID // LANES):
                acc[cur - node_lo, pl.ds(j * LANES, LANES)] = regs[j]

    def chunk(k, carry):
        chunk_lo = base0 + k * CH_V
        off = jnp.minimum(chunk_lo, E - CH_V)
        pltpu.sync_copy(row_hbm.at[pl.ds(off, CH_V)], ridx)
        pltpu.sync_copy(v_hbm.at[pl.ds(off, CH_V)], vbuf)

        def edge(r, carry2):
            cur = carry2[0]
            regs = list(carry2[1:])
            gidx = off + r
            sub = (r // LANES) * LANES
            dst = _lane_extract(ridx[pl.ds(sub, LANES)], r - sub)
            valid = (gidx >= lo) & (gidx < hi) & (gidx >= chunk_lo)
            boundary = valid & (dst != cur)

            @pl.when(boundary)
            def _():
                flush(cur, regs)

            vf = jnp.where(valid, 1.0, 0.0)
            af = jnp.where(boundary, 0.0, 1.0)
            new_regs = []
            for j in range(HID // LANES):
                rj = vbuf[r, pl.ds(j * LANES, LANES)]
                new_regs.append(regs[j] * af + rj * vf)
            new_cur = jnp.where(valid, dst, cur)
            return (new_cur, *new_regs)

        return lax.fori_loop(0, CH_V, edge, carry)

    init = (jnp.int32(-1),) + tuple(
        jnp.zeros((LANES,), jnp.float32) for _ in range(HID // LANES))
    fin = lax.fori_loop(0, nch, chunk, init)
    flush(fin[0], list(fin[1:]))

    @pl.when(wid < NW - 1)
    def _():
        pltpu.sync_copy(acc.at[pl.ds(0, TN)], agg_hbm.at[pl.ds(node_lo, TN)])

    @pl.when(wid == NW - 1)
    def _():
        pltpu.sync_copy(acc.at[pl.ds(0, TN_LAST)],
                        agg_hbm.at[pl.ds(node_lo, TN_LAST)])


def _scatter_seg(v, row_s, ebounds):
    return pl.kernel(
        _scatter_seg_body,
        out_type=jax.ShapeDtypeStruct((N, HID), jnp.float32),
        mesh=_sc_mesh(),
        compiler_params=_SC_PARAMS,
        scratch_types=[
            pltpu.VMEM((CH_V,), jnp.int32),
            pltpu.VMEM((CH_V, HID), jnp.float32),
            pltpu.VMEM((64,), jnp.int32),
            pltpu.VMEM((TN_LAST, HID), jnp.float32),
        ],
    )(v, row_s, ebounds)


def kernel(h, edge_attr, emb_w, emb_b, edge_w1, edge_b1, edge_w2, edge_b2,
           node_w1, node_b1, node_w2, node_b2, out_w, out_b, edge_index):
    row = edge_index[0]
    col = edge_index[1]

    e32 = jnp.arange(E, dtype=jnp.int32)
    row_s, col_s, perm = lax.sort((row, col, e32), dimension=0, num_keys=1,
                                  is_stable=True)
    starts = jnp.concatenate([
        jnp.arange(NW, dtype=jnp.int32) * TN,
        jnp.array([N], dtype=jnp.int32),
    ])
    eb = jnp.searchsorted(row_s, starts, side='left').astype(jnp.int32)
    ebounds = jnp.zeros((64,), jnp.int32).at[:NW + 1].set(eb)

    ea_pad = jnp.pad(edge_attr, ((0, 0), (0, EAP - EDGE_NF)))
    ea_s = _permute_rows(ea_pad, perm)

    w1s = [edge_w1[i, :HID] for i in range(L)]
    w1t = [edge_w1[i, HID:2 * HID] for i in range(L)]
    w1e = [jnp.pad(edge_w1[i, 2 * HID:], ((0, EAP - EDGE_NF), (0, 0)))
           for i in range(L)]
    b1e = [edge_b1[i].reshape(1, HID) for i in range(L)]
    b2e = [edge_b2[i].reshape(1, HID) for i in range(L)]
    nw1x = [node_w1[i, :HID] for i in range(L)]
    nw1a = [node_w1[i, HID:] for i in range(L)]
    nb1 = [node_b1[i].reshape(1, HID) for i in range(L)]
    nb2 = [node_b2[i].reshape(1, HID) for i in range(L)]

    x, xa, xb = _embed_pre(h, emb_w, emb_b.reshape(1, HID),
                           w1s[0], w1t[0], b1e[0])
    for i in range(L):
        g = _gather_add(xa, xb, row_s, col_s)
        v = _edge_mlp(g, ea_s, w1e[i], edge_w2[i], b2e[i])
        agg = _scatter_seg(v, row_s, ebounds)
        if i < L - 1:
            x, xa, xb = _node_pre(x, agg, nw1x[i], nw1a[i], nb1[i],
                                  node_w2[i], nb2[i],
                                  w1s[i + 1], w1t[i + 1], b1e[i + 1])
        else:
            out = _node_final(x, agg, nw1x[i], nw1a[i], nb1[i],
                              node_w2[i], nb2[i], out_w,
                              out_b.reshape(1, OUT_NF))
    return out


# speedup vs baseline: 2.3573x; 1.6650x over previous
import functools

import jax
import jax.numpy as jnp
from jax import lax
from jax.experimental import pallas as pl
from jax.experimental.pallas import tpu as pltpu
from jax.experimental.pallas import tpu_sc as plsc

N = 10000
E = 160000
IN_NF = 128
EDGE_NF = 16
HID = 256
OUT_NF = 128
L = 4
EAP = 128

NC = 2
NS = 16
LANES = 16
NW = NC * NS

BN = 1000
BE = 2000
CH_G = 40
CH_S = 80
HALF = N // NC
ROWS_T = 320
ACC_R = NS * ROWS_T
TRASH = ACC_R - 1


def _silu(x):
    return x * lax.logistic(x)


def _embed_pre_body(h_ref, ew_ref, eb_ref, w1s_ref, w1t_ref, b1_ref,
                    x_ref, xa_ref, xb_ref):
    x = jnp.dot(h_ref[...], ew_ref[...], preferred_element_type=jnp.float32)
    x = x + eb_ref[...]
    x_ref[...] = x
    xa_ref[...] = jnp.dot(x, w1s_ref[...], preferred_element_type=jnp.float32)
    xb_ref[...] = (jnp.dot(x, w1t_ref[...], preferred_element_type=jnp.float32)
                   + b1_ref[...])


def _embed_pre(h, emb_w, emb_b, w1s, w1t, b1):
    grid = (N // BN,)
    full = lambda r, c: pl.BlockSpec((r, c), lambda i: (0, 0))
    return pl.pallas_call(
        _embed_pre_body,
        grid=grid,
        in_specs=[
            pl.BlockSpec((BN, IN_NF), lambda i: (i, 0)),
            full(IN_NF, HID),
            full(1, HID),
            full(HID, HID),
            full(HID, HID),
            full(1, HID),
        ],
        out_specs=[
            pl.BlockSpec((BN, HID), lambda i: (i, 0)),
            pl.BlockSpec((BN, HID), lambda i: (i, 0)),
            pl.BlockSpec((BN, HID), lambda i: (i, 0)),
        ],
        out_shape=[
            jax.ShapeDtypeStruct((N, HID), jnp.float32),
            jax.ShapeDtypeStruct((N, HID), jnp.float32),
            jax.ShapeDtypeStruct((N, HID), jnp.float32),
        ],
    )(h, emb_w, emb_b, w1s, w1t, b1)


def _edge_mlp_body(g_ref, ea_ref, w1e_ref, w2_ref, b2_ref, v_ref):
    u = g_ref[...] + jnp.dot(ea_ref[...], w1e_ref[...],
                             preferred_element_type=jnp.float32)
    u = _silu(u)
    v = jnp.dot(u, w2_ref[...], preferred_element_type=jnp.float32) + b2_ref[...]
    v_ref[...] = _silu(v)


def _edge_mlp(g, edge_attr, w1e, w2, b2):
    grid = (E // BE,)
    full = lambda r, c: pl.BlockSpec((r, c), lambda i: (0, 0))
    return pl.pallas_call(
        _edge_mlp_body,
        grid=grid,
        in_specs=[
            pl.BlockSpec((BE, HID), lambda i: (i, 0)),
            pl.BlockSpec((BE, EAP), lambda i: (i, 0)),
            full(EAP, HID),
            full(HID, HID),
            full(1, HID),
        ],
        out_specs=pl.BlockSpec((BE, HID), lambda i: (i, 0)),
        out_shape=jax.ShapeDtypeStruct((E, HID), jnp.float32),
    )(g, edge_attr, w1e, w2, b2)


def _node_pre_body(x_ref, agg_ref, w1x_ref, w1a_ref, b1_ref, w2_ref, b2_ref,
                   w1s_ref, w1t_ref, b1e_ref, xn_ref, xa_ref, xb_ref):
    u = (jnp.dot(x_ref[...], w1x_ref[...], preferred_element_type=jnp.float32)
         + jnp.dot(agg_ref[...], w1a_ref[...], preferred_element_type=jnp.float32)
         + b1_ref[...])
    u = _silu(u)
    o = jnp.dot(u, w2_ref[...], preferred_element_type=jnp.float32) + b2_ref[...]
    xn = x_ref[...] + o
    xn_ref[...] = xn
    xa_ref[...] = jnp.dot(xn, w1s_ref[...], preferred_element_type=jnp.float32)
    xb_ref[...] = (jnp.dot(xn, w1t_ref[...], preferred_element_type=jnp.float32)
                   + b1e_ref[...])


def _node_pre(x, agg, w1x, w1a, b1, w2, b2, w1s, w1t, b1e):
    grid = (N // BN,)
    full = lambda r, c: pl.BlockSpec((r, c), lambda i: (0, 0))
    return pl.pallas_call(
        _node_pre_body,
        grid=grid,
        in_specs=[
            pl.BlockSpec((BN, HID), lambda i: (i, 0)),
            pl.BlockSpec((BN, HID), lambda i: (i, 0)),
            full(HID, HID), full(HID, HID), full(1, HID),
            full(HID, HID), full(1, HID),
            full(HID, HID), full(HID, HID), full(1, HID),
        ],
        out_specs=[
            pl.BlockSpec((BN, HID), lambda i: (i, 0)),
            pl.BlockSpec((BN, HID), lambda i: (i, 0)),
            pl.BlockSpec((BN, HID), lambda i: (i, 0)),
        ],
        out_shape=[
            jax.ShapeDtypeStruct((N, HID), jnp.float32),
            jax.ShapeDtypeStruct((N, HID), jnp.float32),
            jax.ShapeDtypeStruct((N, HID), jnp.float32),
        ],
    )(x, agg, w1x, w1a, b1, w2, b2, w1s, w1t, b1e)


def _node_final_body(x_ref, agg_ref, w1x_ref, w1a_ref, b1_ref, w2_ref, b2_ref,
                     ow_ref, ob_ref, out_ref):
    u = (jnp.dot(x_ref[...], w1x_ref[...], preferred_element_type=jnp.float32)
         + jnp.dot(agg_ref[...], w1a_ref[...], preferred_element_type=jnp.float32)
         + b1_ref[...])
    u = _silu(u)
    o = jnp.dot(u, w2_ref[...], preferred_element_type=jnp.float32) + b2_ref[...]
    xn = x_ref[...] + o
    out_ref[...] = (jnp.dot(xn, ow_ref[...], preferred_element_type=jnp.float32)
                    + ob_ref[...])


def _node_final(x, agg, w1x, w1a, b1, w2, b2, out_w, out_b):
    grid = (N // BN,)
    full = lambda r, c: pl.BlockSpec((r, c), lambda i: (0, 0))
    return pl.pallas_call(
        _node_final_body,
        grid=grid,
        in_specs=[
            pl.BlockSpec((BN, HID), lambda i: (i, 0)),
            pl.BlockSpec((BN, HID), lambda i: (i, 0)),
            full(HID, HID), full(HID, HID), full(1, HID),
            full(HID, HID), full(1, HID),
            full(HID, OUT_NF), full(1, OUT_NF),
        ],
        out_specs=pl.BlockSpec((BN, OUT_NF), lambda i: (i, 0)),
        out_shape=jax.ShapeDtypeStruct((N, OUT_NF), jnp.float32),
    )(x, agg, w1x, w1a, b1, w2, b2, out_w, out_b)


def _sc_mesh():
    return plsc.VectorSubcoreMesh(core_axis_name="c", subcore_axis_name="s",
                                  num_cores=NC, num_subcores=NS)


_SC_PARAMS = pltpu.CompilerParams(needs_layout_passes=False)


CH_GA = 80
NCH_GA = E // CH_GA


def _gather_add_body(xa_hbm, xb_hbm, rc_hbm, g_hbm,
                     rc0, a0, b0, rc1, a1, b1, sem):
    wid = lax.axis_index("c") * NS + lax.axis_index("s")
    nch = (NCH_GA - wid + NW - 1) // NW

    def start(k, rcb, abuf, bbuf):
        chunk = wid + k * NW
        pltpu.sync_copy(rc_hbm.at[pl.ds(chunk * 2 * CH_GA, 2 * CH_GA)], rcb)
        pltpu.async_copy(xa_hbm.at[rcb.at[pl.ds(0, CH_GA)]], abuf, sem)
        pltpu.async_copy(xb_hbm.at[rcb.at[pl.ds(CH_GA, CH_GA)]], bbuf, sem)

    def finish(k, rcb, abuf, bbuf):
        pltpu.make_async_copy(xa_hbm.at[rcb.at[pl.ds(0, CH_GA)]], abuf,
                              sem).wait()
        pltpu.make_async_copy(xb_hbm.at[rcb.at[pl.ds(CH_GA, CH_GA)]], bbuf,
                              sem).wait()

        def add_row(r, c2):
            for j in range(HID // LANES):
                sl = pl.ds(j * LANES, LANES)
                abuf[r, sl] = abuf[r, sl] + bbuf[r, sl]
            return c2

        lax.fori_loop(0, CH_GA, add_row, 0)
        off = (wid + k * NW) * CH_GA
        pltpu.sync_copy(abuf, g_hbm.at[pl.ds(off, CH_GA)])

    sets = ((rc0, a0, b0), (rc1, a1, b1))

    @pl.when(nch > 0)
    def _():
        start(0, *sets[0])

    def body(k2, carry):
        k = 2 * k2

        @pl.when(k + 1 < nch)
        def _():
            start(k + 1, *sets[1])

        finish(k, *sets[0])

        @pl.when(k + 2 < nch)
        def _():
            start(k + 2, *sets[0])

        @pl.when(k + 1 < nch)
        def _():
            finish(k + 1, *sets[1])

        return carry

    lax.fori_loop(0, (nch + 1) // 2, body, 0)


def _gather_add(xa, xb, rc):
    return pl.kernel(
        _gather_add_body,
        out_type=jax.ShapeDtypeStruct((E, HID), jnp.float32),
        mesh=_sc_mesh(),
        compiler_params=_SC_PARAMS,
        scratch_types=[
            pltpu.VMEM((2 * CH_GA,), jnp.int32),
            pltpu.VMEM((CH_GA, HID), jnp.float32),
            pltpu.VMEM((CH_GA, HID), jnp.float32),
            pltpu.VMEM((2 * CH_GA,), jnp.int32),
            pltpu.VMEM((CH_GA, HID), jnp.float32),
            pltpu.VMEM((CH_GA, HID), jnp.float32),
            pltpu.SemaphoreType.DMA,
        ],
    )(xa, xb, rc)


def _permute_body(ea_hbm, perm_hbm, out_hbm, pidx, ebuf, sem):
    wid = lax.axis_index("c") * NS + lax.axis_index("s")
    epw = E // NW
    base = wid * epw

    def body(ei, carry):
        off = base + ei * CH_G
        pltpu.sync_copy(perm_hbm.at[pl.ds(off, CH_G)], pidx)
        pltpu.async_copy(ea_hbm.at[pidx], ebuf, sem).wait()
        pltpu.sync_copy(ebuf, out_hbm.at[pl.ds(off, CH_G)])
        return carry

    lax.fori_loop(0, epw // CH_G, body, 0)


def _permute_rows(edge_attr, perm):
    return pl.kernel(
        _permute_body,
        out_type=jax.ShapeDtypeStruct((E, EAP), jnp.float32),
        mesh=_sc_mesh(),
        compiler_params=_SC_PARAMS,
        scratch_types=[
            pltpu.VMEM((CH_G,), jnp.int32),
            pltpu.VMEM((CH_G, EAP), jnp.float32),
            pltpu.SemaphoreType.DMA,
        ],
    )(edge_attr, perm)


TN = 312
TN_LAST = N - (NW - 1) * TN
CH_V = 64


def _lane_extract(vec, lane):
    io = lax.iota(jnp.int32, LANES)
    return jnp.sum(jnp.where(io == lane, vec, 0))


def _scatter_seg_body(v_hbm, row_hbm, eb_hbm, agg_hbm,
                      r0, v0, r1, v1, bvec, acc, sem):
    wid = lax.axis_index("c") * NS + lax.axis_index("s")
    node_lo = wid * TN

    pltpu.sync_copy(eb_hbm, bvec)

    def bound(i):
        sub = (i // LANES) * LANES
        return _lane_extract(bvec[pl.ds(sub, LANES)], i - sub)

    lo = bound(wid)
    hi = bound(wid + 1)

    zz = jnp.zeros((LANES,), jnp.float32)

    def zrow(r, c2):
        for j in range(HID // LANES):
            acc[r, pl.ds(j * LANES, LANES)] = zz
        return c2

    lax.fori_loop(0, TN_LAST, zrow, 0)

    base0 = (lo // 8) * 8
    nch = (hi - base0 + CH_V - 1) // CH_V
    npair = (nch + 1) // 2

    def flush(cond, cur, regs):
        @pl.when(cond & (cur >= 0))
        def _():
            for j in range(HID // LANES):
                acc[cur - node_lo, pl.ds(j * LANES, LANES)] = regs[j]

    def chunk_off(k):
        return jnp.minimum(base0 + k * CH_V, E - CH_V)

    def start(k, ridx, vbuf):
        off = chunk_off(k)
        pltpu.async_copy(row_hbm.at[pl.ds(off, CH_V)], ridx, sem)
        pltpu.async_copy(v_hbm.at[pl.ds(off, CH_V)], vbuf, sem)

    def process(k, ridx, vbuf, carry):
        off = chunk_off(k)
        chunk_lo = base0 + k * CH_V
        pltpu.make_async_copy(row_hbm.at[pl.ds(off, CH_V)], ridx, sem).wait()
        pltpu.make_async_copy(v_hbm.at[pl.ds(off, CH_V)], vbuf, sem).wait()
        io = lax.iota(jnp.int32, LANES)

        def group(gi, c2):
            cur = c2[0]
            regs = list(c2[1:])
            subvec = ridx[pl.ds(gi * LANES, LANES)]
            for l in range(LANES):
                r = gi * LANES + l
                gidx = off + r
                dst = jnp.sum(jnp.where(io == l, subvec, 0))
                valid = (gidx >= lo) & (gidx < hi) & (gidx >= chunk_lo)
                boundary = valid & (dst != cur)
                flush(boundary, cur, regs)
                vf = jnp.where(valid, 1.0, 0.0)
                af = jnp.where(boundary, 0.0, 1.0)
                regs = [regs[j] * af + vbuf[r, pl.ds(j * LANES, LANES)] * vf
                        for j in range(HID // LANES)]
                cur = jnp.where(valid, dst, cur)
            return (cur, *regs)

        return lax.fori_loop(0, CH_V // LANES, group, carry)

    init = (jnp.int32(-1),) + tuple(
        jnp.zeros((LANES,), jnp.float32) for _ in range(HID // LANES))

    @pl.when(npair > 0)
    def _():
        start(0, r0, v0)

    def body(k2, carry):
        k = 2 * k2
        start(k + 1, r1, v1)
        carry = process(k, r0, v0, carry)

        @pl.when(k + 2 < 2 * npair)
        def _():
            start(k + 2, r0, v0)

        carry = process(k + 1, r1, v1, carry)
        return carry

    fin = lax.fori_loop(0, npair, body, init)
    flush(jnp.bool_(True), fin[0], list(fin[1:]))

    @pl.when(wid < NW - 1)
    def _():
        pltpu.sync_copy(acc.at[pl.ds(0, TN)], agg_hbm.at[pl.ds(node_lo, TN)])

    @pl.when(wid == NW - 1)
    def _():
        pltpu.sync_copy(acc.at[pl.ds(0, TN_LAST)],
                        agg_hbm.at[pl.ds(node_lo, TN_LAST)])


def _scatter_seg(v, row_s, ebounds):
    return pl.kernel(
        _scatter_seg_body,
        out_type=jax.ShapeDtypeStruct((N, HID), jnp.float32),
        mesh=_sc_mesh(),
        compiler_params=_SC_PARAMS,
        scratch_types=[
            pltpu.VMEM((CH_V,), jnp.int32),
            pltpu.VMEM((CH_V, HID), jnp.float32),
            pltpu.VMEM((CH_V,), jnp.int32),
            pltpu.VMEM((CH_V, HID), jnp.float32),
            pltpu.VMEM((64,), jnp.int32),
            pltpu.VMEM((TN_LAST, HID), jnp.float32),
            pltpu.SemaphoreType.DMA,
        ],
    )(v, row_s, ebounds)


def kernel(h, edge_attr, emb_w, emb_b, edge_w1, edge_b1, edge_w2, edge_b2,
           node_w1, node_b1, node_w2, node_b2, out_w, out_b, edge_index):
    row = edge_index[0]
    col = edge_index[1]

    e32 = jnp.arange(E, dtype=jnp.int32)
    row_s, col_s, perm = lax.sort((row, col, e32), dimension=0, num_keys=1,
                                  is_stable=True)
    starts = jnp.concatenate([
        jnp.arange(NW, dtype=jnp.int32) * TN,
        jnp.array([N], dtype=jnp.int32),
    ])
    eb = jnp.searchsorted(row_s, starts, side='left').astype(jnp.int32)
    ebounds = jnp.zeros((64,), jnp.int32).at[:NW + 1].set(eb)

    ea_pad = jnp.pad(edge_attr, ((0, 0), (0, EAP - EDGE_NF)))
    ea_s = _permute_rows(ea_pad, perm)

    rc = jnp.stack([row_s.reshape(NCH_GA, CH_GA),
                    col_s.reshape(NCH_GA, CH_GA)], axis=1).reshape(-1)

    w1s = [edge_w1[i, :HID] for i in range(L)]
    w1t = [edge_w1[i, HID:2 * HID] for i in range(L)]
    w1e = [jnp.pad(edge_w1[i, 2 * HID:], ((0, EAP - EDGE_NF), (0, 0)))
           for i in range(L)]
    b1e = [edge_b1[i].reshape(1, HID) for i in range(L)]
    b2e = [edge_b2[i].reshape(1, HID) for i in range(L)]
    nw1x = [node_w1[i, :HID] for i in range(L)]
    nw1a = [node_w1[i, HID:] for i in range(L)]
    nb1 = [node_b1[i].reshape(1, HID) for i in range(L)]
    nb2 = [node_b2[i].reshape(1, HID) for i in range(L)]

    x, xa, xb = _embed_pre(h, emb_w, emb_b.reshape(1, HID),
                           w1s[0], w1t[0], b1e[0])
    for i in range(L):
        g = _gather_add(xa, xb, rc)
        v = _edge_mlp(g, ea_s, w1e[i], edge_w2[i], b2e[i])
        agg = _scatter_seg(v, row_s, ebounds)
        if i < L - 1:
            x, xa, xb = _node_pre(x, agg, nw1x[i], nw1a[i], nb1[i],
                                  node_w2[i], nb2[i],
                                  w1s[i + 1], w1t[i + 1], b1e[i + 1])
        else:
            out = _node_final(x, agg, nw1x[i], nw1a[i], nb1[i],
                              node_w2[i], nb2[i], out_w,
                              out_b.reshape(1, OUT_NF))
    return out
